# baseline (speedup 1.0000x reference)
import dataclasses

import jax
import jax.numpy as jnp
from jax import lax
from jax.experimental import pallas as pl
from jax.experimental.pallas import tpu as pltpu
from jax.experimental.pallas import tpu_sc as plsc

D = 4096
NSC = 64
NTILES = 16
L = 16
F = D // NTILES
NCH = F // L
NG = NSC // L
TOPK = int(0.15 * D)
STATS = 2 * NSC + L

RADIX_BITS = 6
RADIX_BINS = 1 << RADIX_BITS
NPASS = 5


def _router_body(h_hbm, g_hbm, p_hbm, m_hbm, wg_hbm, bg_hbm,
                 out_hbm, w_hbm,
                 hbuf, Pbuf, Mbuf, wgbuf, gbuf, bgbuf,
                 statsbuf, statsall, statsg, wvec, vbuf, bbuf,
                 hist2d, histg, histall, tierow, tieb, outbuf,
                 rbstats, rbhist, rbtie,
                 sh_stats, sh_hist0, sh_hist1, sh_hist2, sh_hist3, sh_hist4,
                 sh_tie):
    sh_hists = (sh_hist0, sh_hist1, sh_hist2, sh_hist3, sh_hist4)
    t = lax.axis_index("s")

    def bf16_round(x):
        b = plsc.bitcast(x, jnp.int32)
        r = b + jnp.int32(0x7FFF) + jnp.bitwise_and(
            lax.shift_right_logical(b, 16), 1)
        return plsc.bitcast(jnp.bitwise_and(r, jnp.int32(-65536)), jnp.float32)

    def _csum_chunks(get, nch_):
        acc = jnp.zeros((L,), jnp.int32)
        for j in range(nch_):
            x = get(j)
            if x.dtype == jnp.float32:
                x = plsc.bitcast(x, jnp.int32)
            acc = acc + x
        return jnp.sum(acc)

    def publish(src_ref, sh_row, rb_ref, nwords, magic):
        tag = _csum_chunks(lambda j: src_ref[pl.ds(j * L, L)], nwords // L)
        tagv = jnp.where(lane == 0, tag + jnp.int32(magic), 0)
        if src_ref.dtype == jnp.float32:
            tagv = plsc.bitcast(tagv, jnp.float32)
        src_ref[pl.ds(nwords, L)] = tagv
        pltpu.sync_copy(src_ref, sh_row)

        def check():
            good = None
            for j in range(nwords // L + 1):
                a = rb_ref[pl.ds(j * L, L)]
                b = src_ref[pl.ds(j * L, L)]
                if a.dtype == jnp.float32:
                    a = plsc.bitcast(a, jnp.int32)
                    b = plsc.bitcast(b, jnp.int32)
                g = jnp.all(a == b)
                good = g if good is None else jnp.logical_and(good, g)
            return good

        pltpu.sync_copy(sh_row, rb_ref)

        def retry(it, ok):
            @pl.when(jnp.logical_not(ok))
            def _():
                pltpu.sync_copy(sh_row, rb_ref)

            return jnp.logical_or(ok, check())

        pl.loop(0, 3, init_carry=check())(retry)

    def consume(sh_buf, dst_ref, nwords, magic):
        def check():
            colv = jnp.full((L,), nwords, jnp.int32)
            tags = plsc.load_gather(dst_ref, [lane, colv])
            if tags.dtype == jnp.float32:
                tags = plsc.bitcast(tags, jnp.int32)
            exp = izero
            for r in range(NTILES):
                cs = _csum_chunks(lambda j: dst_ref[r, pl.ds(j * L, L)],
                                  nwords // L)
                exp = jnp.where(lane == r, cs + jnp.int32(magic), exp)
            return jnp.all(tags == exp)

        pltpu.sync_copy(sh_buf, dst_ref)

        def retry(it, ok):
            @pl.when(jnp.logical_not(ok))
            def _():
                pltpu.sync_copy(sh_buf, dst_ref)

            return jnp.logical_or(ok, check())

        pl.loop(0, 3, init_carry=check())(retry)
    lane = lax.iota(jnp.int32, L)
    fzero = jnp.zeros((L,), jnp.float32)
    izero = jnp.zeros((L,), jnp.int32)
    ione = jnp.ones((L,), jnp.int32)

    base = t * F
    pltpu.sync_copy(h_hbm.at[pl.ds(base, F)], hbuf)
    pltpu.sync_copy(p_hbm.at[:, pl.ds(base, F)], Pbuf)
    pltpu.sync_copy(m_hbm.at[:, pl.ds(base, F)], Mbuf)
    pltpu.sync_copy(wg_hbm, wgbuf)
    pltpu.sync_copy(g_hbm, gbuf)
    pltpu.sync_copy(bg_hbm, bgbuf)

    hs = [hbuf[pl.ds(c * L, L)] for c in range(NCH)]
    hhv = fzero
    for c in range(NCH):
        hhv = hhv + hs[c] * hs[c]
    hh_part = jnp.sum(hhv)
    statsbuf[pl.ds(2 * NSC, L)] = jnp.where(lane == 0, hh_part, 0.0)

    def scene_body(i, carry):
        dvec, pvec = carry
        dp = fzero
        pp = fzero
        for c in range(NCH):
            prow = Pbuf[i, pl.ds(c * L, L)]
            dp = dp + hs[c] * prow
            pp = pp + prow * prow
        dsum = jnp.sum(dp)
        psum = jnp.sum(pp)
        lpos = jnp.bitwise_and(i, L - 1)
        dvec = jnp.where(lane == lpos, dsum, dvec)
        pvec = jnp.where(lane == lpos, psum, pvec)

        @pl.when(lpos == L - 1)
        def _():
            statsbuf[pl.ds(i - (L - 1), L)] = dvec
            statsbuf[pl.ds(NSC + i - (L - 1), L)] = pvec

        reset = lpos == L - 1
        dvec = jnp.where(reset, fzero, dvec)
        pvec = jnp.where(reset, fzero, pvec)
        return dvec, pvec

    pl.loop(0, NSC, init_carry=(fzero, fzero))(scene_body)

    publish(statsbuf, sh_stats.at[t], rbstats, STATS, 0x13570000)
    plsc.subcore_barrier()
    consume(sh_stats, statsall, STATS, 0x13570000)
    for j in range(STATS // L):
        acc = fzero
        for r in range(NTILES):
            acc = acc + statsall[r, pl.ds(j * L, L)]
        statsg[pl.ds(j * L, L)] = acc

    gv = bf16_round(gbuf[pl.ds(0, L)])
    gsc = [gv[j] for j in range(8)]
    hh = jnp.sum(statsg[pl.ds(2 * NSC, L)])
    logits = []
    for g in range(NG):
        acc = bgbuf[pl.ds(g * L, L)]
        rowbase = (g * L + lane) * 8
        for j in range(8):
            col = bf16_round(plsc.load_gather(wgbuf, [rowbase + j]))
            acc = acc + gsc[j] * col
        dots = statsg[pl.ds(g * L, L)]
        pps = statsg[pl.ds(NSC + g * L, L)]
        x = hh * pps
        xb = plsc.bitcast(x, jnp.int32)
        y = plsc.bitcast(jnp.int32(0x5F3759DF) - lax.shift_right_logical(xb, 1),
                         jnp.float32)
        for _ in range(3):
            y = y * (1.5 - 0.5 * x * y * y)
        s = x * y
        s = 0.5 * (s + x / jnp.maximum(s, 1e-30))
        s = jnp.where(x < 1e-32, 0.0, s)
        sim = dots / jnp.maximum(s, 1e-8)
        logits.append(sim + acc)

    m = logits[0]
    for g in range(1, NG):
        m = jnp.maximum(m, logits[g])
    msc = jnp.max(m)
    es = [jnp.exp(lg - msc) for lg in logits]
    zv = es[0]
    for g in range(1, NG):
        zv = zv + es[g]
    zs_sum = jnp.sum(zv)
    for g in range(NG):
        wvec[pl.ds(g * L, L)] = es[g] / zs_sum

    wregs = [wvec[pl.ds(g * L, L)] for g in range(NG)]
    wsc = [wregs[i // L][i % L] for i in range(NSC)]

    def chunk_body(c):
        sl = pl.ds(c * L, L)
        ps = [fzero] * 8
        for t_ in range(8):
            for s_ in range(8):
                i = t_ * 8 + s_
                ps[s_] = ps[s_] + wsc[i] * Mbuf[i, sl]
        q = [ps[s_] + ps[s_ + 4] for s_ in range(4)]
        r = [q[s_] + q[s_ + 2] for s_ in range(2)]
        z = r[0] + r[1]
        v = 1.0 / (1.0 + jnp.exp(-z))
        vbuf[sl] = v
        bbuf[sl] = plsc.bitcast(v, jnp.int32)

    pl.loop(0, NCH)(chunk_body)

    kk = jnp.int32(TOPK)
    T = jnp.int32(0)
    for p in range(NPASS):
        binshift = 24 - RADIX_BITS * p

        @pl.loop(0, NTILES * RADIX_BINS, step=L)
        def _(o):
            hist2d[pl.ds(o, L)] = izero

        def scat(c, _p=p, _bs=binshift, _T=T):
            bits = bbuf[pl.ds(c * L, L)]
            binv = jnp.bitwise_and(lax.shift_right_logical(bits, _bs),
                                   RADIX_BINS - 1)
            idx = lane * RADIX_BINS + binv
            if _p == 0:
                plsc.addupdate_scatter(hist2d, [idx], ione)
            else:
                act = lax.shift_right_logical(bits, _bs + RADIX_BITS) == _T
                plsc.addupdate_scatter(hist2d, [idx], ione, mask=act)

        pl.loop(0, NCH)(scat)

        for j in range(RADIX_BINS // L):
            acc = izero
            for r in range(NTILES):
                acc = acc + hist2d[pl.ds(r * RADIX_BINS + j * L, L)]
            histg[pl.ds(j * L, L)] = acc

        publish(histg, sh_hists[p].at[t], rbhist, RADIX_BINS, 0x24680000 + p)
        plsc.subcore_barrier()
        consume(sh_hists[p], histall, RADIX_BINS, 0x24680000 + p)
        for j in range(RADIX_BINS // L):
            acc = izero
            for r in range(NTILES):
                acc = acc + histall[r, pl.ds(j * L, L)]
            histg[pl.ds(j * L, L)] = acc

        suffix = jnp.int32(0)
        selbin = jnp.int32(0)
        asel = jnp.int32(0)
        for j in range(RADIX_BINS // L - 1, -1, -1):
            hv = histg[pl.ds(j * L, L)]
            tot = jnp.sum(hv)
            cs = plsc.cumsum(hv)
            above = suffix + (tot - cs)
            found = jnp.logical_and(above < kk, above + hv >= kk)
            selbin = selbin + jnp.sum(jnp.where(found, j * L + lane, 0))
            asel = asel + jnp.sum(jnp.where(found, above, 0))
            suffix = suffix + tot
        kk = kk - asel
        T = T * RADIX_BINS + selbin

    def cnt_body(c, acc):
        bits = bbuf[pl.ds(c * L, L)]
        return acc + jnp.sum(jnp.where(bits == T, ione, izero))

    cnt = pl.loop(0, NCH, init_carry=jnp.int32(0))(cnt_body)
    tierow[pl.ds(0, L)] = jnp.where(lane == 0, cnt, 0)
    publish(tierow, sh_tie.at[t], rbtie, L, 0x369C0000)
    plsc.subcore_barrier()
    consume(sh_tie, tieb, L, 0x369C0000)
    counts = plsc.load_gather(tieb, [lane, izero])
    excl = plsc.cumsum(counts) - counts
    my_off = jnp.sum(jnp.where(lane == t, excl, 0))

    def out_body(c, seen):
        sl = pl.ds(c * L, L)
        bits = bbuf[sl]
        v = vbuf[sl]
        hvl = hbuf[sl]
        eq = bits == T
        eqi = jnp.where(eq, ione, izero)
        incl = plsc.cumsum(eqi)
        rank = my_off + seen + incl - eqi
        keep = jnp.logical_or(bits > T, jnp.logical_and(eq, rank < kk))
        outbuf[sl] = jnp.where(keep, hvl * v, 0.0)
        return seen + jnp.sum(eqi)

    pl.loop(0, NCH, init_carry=jnp.int32(0))(out_body)
    pltpu.sync_copy(outbuf, out_hbm.at[pl.ds(base, F)])

    @pl.when(t == 0)
    def _():
        pltpu.sync_copy(wvec.at[pl.ds(0, NSC)], w_hbm)


@jax.jit
def _router(h, goal16, prototypes, masks, wg_flat, b_goal):
    mesh = plsc.VectorSubcoreMesh(core_axis_name="c", subcore_axis_name="s",
                                  num_cores=1)
    cp = pltpu.CompilerParams()
    if "needs_layout_passes" in pltpu.CompilerParams.__dataclass_fields__:
        cp = dataclasses.replace(cp, needs_layout_passes=False)
    fn = pl.kernel(
        _router_body,
        compiler_params=cp,
        out_type=(jax.ShapeDtypeStruct((D,), jnp.float32),
                  jax.ShapeDtypeStruct((NSC,), jnp.float32)),
        mesh=mesh,
        scratch_types=[
            pltpu.VMEM((F,), jnp.float32),
            pltpu.VMEM((NSC, F), jnp.float32),
            pltpu.VMEM((NSC, F), jnp.float32),
            pltpu.VMEM((NSC * 8,), jnp.float32),
            pltpu.VMEM((16,), jnp.float32),
            pltpu.VMEM((NSC,), jnp.float32),
            pltpu.VMEM((STATS + L,), jnp.float32),
            pltpu.VMEM((NTILES, STATS + L), jnp.float32),
            pltpu.VMEM((STATS,), jnp.float32),
            pltpu.VMEM((NSC + L,), jnp.float32),
            pltpu.VMEM((F,), jnp.float32),
            pltpu.VMEM((F,), jnp.int32),
            pltpu.VMEM((NTILES * RADIX_BINS,), jnp.int32),
            pltpu.VMEM((RADIX_BINS + L,), jnp.int32),
            pltpu.VMEM((NTILES, RADIX_BINS + L), jnp.int32),
            pltpu.VMEM((2 * L,), jnp.int32),
            pltpu.VMEM((NTILES, 2 * L), jnp.int32),
            pltpu.VMEM((F,), jnp.float32),
            pltpu.VMEM((STATS + L,), jnp.float32),
            pltpu.VMEM((RADIX_BINS + L,), jnp.int32),
            pltpu.VMEM((2 * L,), jnp.int32),
            pltpu.VMEM_SHARED((NTILES, STATS + L), jnp.float32),
            pltpu.VMEM_SHARED((NTILES, RADIX_BINS + L), jnp.int32),
            pltpu.VMEM_SHARED((NTILES, RADIX_BINS + L), jnp.int32),
            pltpu.VMEM_SHARED((NTILES, RADIX_BINS + L), jnp.int32),
            pltpu.VMEM_SHARED((NTILES, RADIX_BINS + L), jnp.int32),
            pltpu.VMEM_SHARED((NTILES, RADIX_BINS + L), jnp.int32),
            pltpu.VMEM_SHARED((NTILES, 2 * L), jnp.int32),
        ],
    )
    return fn(h, goal16, prototypes, masks, wg_flat, b_goal)


def kernel(h, goal, prototypes, masks, W_goal, b_goal):
    goal16 = jnp.concatenate([goal, jnp.zeros((8,), jnp.float32)])
    wg_flat = W_goal.reshape(-1)
    return _router(h, goal16, prototypes, masks, wg_flat, b_goal)
